# baseline (speedup 1.0000x reference)
import functools

import jax
import jax.numpy as jnp
from jax.experimental import pallas as pl
from jax.experimental.pallas import tpu as pltpu

_EPS_BN = 1e-5
_EPS_DEN = 1e-20


def _node_init_body(coord_ref, wn_ref, ve_ref, un_ref, vn_ref,
                    bve_ref, bun_ref, bvn_ref,
                    x_ref, vxe_ref, ux_ref, vx_ref):
    f32 = jnp.float32
    coord = coord_ref[...].astype(jnp.bfloat16).astype(f32)
    wn = wn_ref[...].astype(jnp.bfloat16).astype(f32)
    nd = coord.shape[1]
    x = coord[:, 0:1] * wn[0][None, :]
    for d in range(1, nd):
        x = x + coord[:, d:d + 1] * wn[d][None, :]
    x_ref[...] = x
    x16 = x.astype(jnp.bfloat16)
    vxe_ref[...] = jnp.dot(x16, ve_ref[...], preferred_element_type=f32) + bve_ref[...]
    ux_ref[...] = jnp.dot(x16, un_ref[...], preferred_element_type=f32) + bun_ref[...]
    vx_ref[...] = jnp.dot(x16, vn_ref[...], preferred_element_type=f32) + bvn_ref[...]


def _node_update_core(x_ref, ux_ref, agg_ref, den_ref, gn_ref, bn_ref):
    t = ux_ref[...] + agg_ref[...] / (den_ref[...] + _EPS_DEN)
    mu = jnp.mean(t, axis=0, keepdims=True)
    var = jnp.mean((t - mu) ** 2, axis=0, keepdims=True)
    h = gn_ref[...] * (t - mu) * jax.lax.rsqrt(var + _EPS_BN) + bn_ref[...]
    return x_ref[...] + jnp.maximum(h, 0.0)


def _node_update_proj_body(x_ref, ux_ref, agg_ref, den_ref, gn_ref, bn_ref,
                           ve_ref, un_ref, vn_ref, bve_ref, bun_ref, bvn_ref,
                           xo_ref, vxe_ref, uxo_ref, vxo_ref):
    x = _node_update_core(x_ref, ux_ref, agg_ref, den_ref, gn_ref, bn_ref)
    xo_ref[...] = x
    f32 = jnp.float32
    x16 = x.astype(jnp.bfloat16)
    vxe_ref[...] = jnp.dot(x16, ve_ref[...], preferred_element_type=f32) + bve_ref[...]
    uxo_ref[...] = jnp.dot(x16, un_ref[...], preferred_element_type=f32) + bun_ref[...]
    vxo_ref[...] = jnp.dot(x16, vn_ref[...], preferred_element_type=f32) + bvn_ref[...]


def _node_update_last_body(x_ref, ux_ref, agg_ref, den_ref, gn_ref, bn_ref,
                           xo_ref):
    xo_ref[...] = _node_update_core(x_ref, ux_ref, agg_ref, den_ref,
                                    gn_ref, bn_ref)


def _edge_epilogue(etmp, vx_ref, s_ref, agg_ref, den_ref, s1_ref, s2_ref):
    f32 = jnp.float32
    hi = jax.lax.Precision.HIGHEST
    gate = jax.nn.sigmoid(etmp)
    icr, vv, hd = etmp.shape
    s_mat = s_ref[...]
    g2 = gate.reshape(icr * vv, hd)
    m2 = (gate * vx_ref[0][None, :, :]).reshape(icr * vv, hd)
    t2 = etmp.reshape(icr * vv, hd)
    q2 = (etmp * etmp).reshape(icr * vv, hd)
    r_g = jnp.dot(s_mat, g2, precision=hi, preferred_element_type=f32)
    r_m = jnp.dot(s_mat, m2, precision=hi, preferred_element_type=f32)
    r_t = jnp.dot(s_mat, t2, precision=hi, preferred_element_type=f32)
    r_q = jnp.dot(s_mat, q2, precision=hi, preferred_element_type=f32)
    den_ref[0, 0] = r_g[:icr]
    agg_ref[0, 0] = r_m[:icr]
    p1 = r_t[icr:icr + 1]
    p2 = r_q[icr:icr + 1]
    first = (pl.program_id(0) == 0) & (pl.program_id(1) == 0)

    @pl.when(first)
    def _():
        s1_ref[...] = p1
        s2_ref[...] = p2

    @pl.when(jnp.logical_not(first))
    def _():
        s1_ref[...] += p1
        s2_ref[...] += p2


def _edge_matmul(e_tile, u16_ref, bias_ref, vxe_ref, i0, ic_rows):
    icr, vv, hd = e_tile.shape
    m2 = jnp.dot(e_tile.reshape(icr * vv, hd).astype(jnp.bfloat16),
                 u16_ref[...],
                 preferred_element_type=jnp.float32).reshape(icr, vv, hd)
    vxe_i = vxe_ref[0, pl.ds(i0, ic_rows), :]
    vxe_j = vxe_ref[0] + bias_ref[...]
    return m2 + (vxe_i[:, None, :] + vxe_j[None, :, :])


def _bn_residual(e_tile, etmp, s1_ref, s2_ref, ge_ref, bte_ref, m_total):
    mu = s1_ref[0] / m_total
    var = s2_ref[0] / m_total - mu * mu
    inv = ge_ref[0] * jax.lax.rsqrt(var + _EPS_BN)
    shift = bte_ref[0] - mu * inv
    return e_tile + jnp.maximum(
        etmp * inv[None, None, :] + shift[None, None, :], 0.0)


def _edge_first_body(vals_ref, tags_ref, u16_ref, bias_ref, wvp_ref, ptab_ref,
                     vxe_ref, vx_ref, s_ref,
                     e_ref, agg_ref, den_ref, s1_ref, s2_ref,
                     *, ic_rows, voc):
    i0 = pl.program_id(1) * ic_rows
    vals = vals_ref[0, pl.ds(i0, ic_rows), :]
    ncat = tags_ref.shape[3]
    e0 = vals[:, :, None] * wvp_ref[0][None, None, :]
    for c in range(ncat):
        t = tags_ref[0, pl.ds(i0, ic_rows), :, c]
        for v in range(voc):
            m = (t == v)[:, :, None]
            e0 = e0 + jnp.where(m, ptab_ref[c * voc + v][None, None, :], 0.0)
    e_ref[0] = e0
    etmp = _edge_matmul(e0, u16_ref, bias_ref, vxe_ref, i0, ic_rows)
    _edge_epilogue(etmp, vx_ref, s_ref, agg_ref, den_ref, s1_ref, s2_ref)


def _edge_mid_body(e_ref, s1_ref, s2_ref, ge_ref, bte_ref,
                   up_ref, biasp_ref, vxep_ref,
                   uc_ref, biasc_ref, vxec_ref, vx_ref, s_ref,
                   eo_ref, agg_ref, den_ref, s1o_ref, s2o_ref,
                   *, ic_rows, m_total):
    i0 = pl.program_id(1) * ic_rows
    e_prev = e_ref[0]
    etmp_prev = _edge_matmul(e_prev, up_ref, biasp_ref, vxep_ref, i0, ic_rows)
    e_new = _bn_residual(e_prev, etmp_prev, s1_ref, s2_ref, ge_ref, bte_ref,
                         m_total)
    eo_ref[0] = e_new
    etmp = _edge_matmul(e_new, uc_ref, biasc_ref, vxec_ref, i0, ic_rows)
    _edge_epilogue(etmp, vx_ref, s_ref, agg_ref, den_ref, s1o_ref, s2o_ref)


def _edge_final_body(e_ref, s1_ref, s2_ref, ge_ref, bte_ref,
                     up_ref, biasp_ref, vxep_ref,
                     eo_ref, *, ic_rows, m_total):
    i0 = pl.program_id(1) * ic_rows
    e_prev = e_ref[0]
    etmp_prev = _edge_matmul(e_prev, up_ref, biasp_ref, vxep_ref, i0, ic_rows)
    eo_ref[0] = _bn_residual(e_prev, etmp_prev, s1_ref, s2_ref, ge_ref,
                             bte_ref, m_total)


def kernel(x_edges, x_edges_values, x_nodes_coord, W_nodes, W_vals, E_cat,
           Ue, Ve, Un, Vn, be, bve, bun, bvn,
           gamma_n, beta_n, gamma_e, beta_e):
    B, V, _, ncat = x_edges.shape
    H = W_nodes.shape[1]
    L = Ue.shape[0]
    voc = E_cat.shape[1]
    Hh = W_vals.shape[1]
    N = B * V
    f32 = jnp.float32

    IC = next(c for c in (32, 25, 20, 16, 10, 8, 5, 4, 2, 1) if V % c == 0)
    NIC = V // IC
    m_total = float(B * V * V)

    vals = x_edges_values.astype(f32)
    tags = x_edges.astype(jnp.int32)

    wvp = jnp.pad(W_vals[0], (0, H - Hh))[None, :]
    ptab = jnp.pad(E_cat, ((0, 0), (0, 0), (Hh, 0))).reshape(ncat * voc, H)
    Ue16 = Ue.astype(jnp.bfloat16)
    Ve16 = Ve.astype(jnp.bfloat16)
    SR = ((IC + 1 + 7) // 8) * 8
    rows = jnp.arange(SR)[:, None]
    cols = jnp.arange(IC * V)[None, :]
    s_mat = ((cols // V == rows) | (rows == IC)).astype(f32)
    Un16 = Un.astype(jnp.bfloat16)
    Vn16 = Vn.astype(jnp.bfloat16)

    cparams = pltpu.CompilerParams(
        dimension_semantics=("arbitrary", "arbitrary"))

    node_init = pl.pallas_call(
        _node_init_body,
        out_shape=[jax.ShapeDtypeStruct((N, H), f32)] * 4,
    )
    node_update_proj = pl.pallas_call(
        _node_update_proj_body,
        out_shape=[jax.ShapeDtypeStruct((N, H), f32)] * 4,
    )
    node_update_last = pl.pallas_call(
        _node_update_last_body,
        out_shape=jax.ShapeDtypeStruct((N, H), f32),
    )

    _rep = lambda b, i: (0, 0)
    _bvh = pl.BlockSpec((1, V, H), lambda b, i: (b, 0, 0))
    _tile = pl.BlockSpec((1, IC, V, H), lambda b, i: (b, i, 0, 0))
    _aggspec = pl.BlockSpec((1, 1, IC, H), lambda b, i: (b, i, 0, 0))
    _row = pl.BlockSpec((1, H), _rep)
    _wspec = pl.BlockSpec((H, H), _rep)
    edge_out_shapes = [
        jax.ShapeDtypeStruct((B, V, V, H), f32),
        jax.ShapeDtypeStruct((B, NIC, IC, H), f32),
        jax.ShapeDtypeStruct((B, NIC, IC, H), f32),
        jax.ShapeDtypeStruct((1, H), f32),
        jax.ShapeDtypeStruct((1, H), f32),
    ]
    edge_out_specs = [_tile, _aggspec, _aggspec, _row, _row]

    edge_first = pl.pallas_call(
        functools.partial(_edge_first_body, ic_rows=IC, voc=voc),
        grid=(B, NIC),
        in_specs=[
            pl.BlockSpec((1, V, V), lambda b, i: (b, 0, 0)),
            pl.BlockSpec((1, V, V, ncat), lambda b, i: (b, 0, 0, 0)),
            _wspec,
            _row,
            _row,
            pl.BlockSpec((ncat * voc, H), _rep),
            _bvh, _bvh,
            pl.BlockSpec((SR, IC * V), _rep),
        ],
        out_specs=edge_out_specs,
        out_shape=edge_out_shapes,
        compiler_params=cparams,
    )
    edge_mid = pl.pallas_call(
        functools.partial(_edge_mid_body, ic_rows=IC, m_total=m_total),
        grid=(B, NIC),
        in_specs=[_tile, _row, _row, _row, _row,
                  _wspec, _row, _bvh,
                  _wspec, _row, _bvh, _bvh,
                  pl.BlockSpec((SR, IC * V), _rep)],
        out_specs=edge_out_specs,
        out_shape=edge_out_shapes,
        compiler_params=cparams,
    )
    edge_final = pl.pallas_call(
        functools.partial(_edge_final_body, ic_rows=IC, m_total=m_total),
        grid=(B, NIC),
        in_specs=[_tile, _row, _row, _row, _row,
                  _wspec, _row, _bvh],
        out_specs=_tile,
        out_shape=jax.ShapeDtypeStruct((B, V, V, H), f32),
        compiler_params=cparams,
    )

    coord = x_nodes_coord.reshape(N, -1).astype(f32)
    x, vxe, ux, vx = node_init(
        coord, W_nodes, Ve16[0], Un16[0], Vn16[0],
        bve[0][None], bun[0][None], bvn[0][None])

    vxe_prev = vxe.reshape(B, V, H)
    e_state, agg4, den4, s1, s2 = edge_first(
        vals, tags, Ue16[0], be[0][None], wvp, ptab,
        vxe_prev, vx.reshape(B, V, H), s_mat)

    for l in range(L):
        agg = agg4.reshape(N, H)
        den = den4.reshape(N, H)
        if l < L - 1:
            x, vxe, ux, vx = node_update_proj(
                x, ux, agg, den, gamma_n[l][None], beta_n[l][None],
                Ve16[l + 1], Un16[l + 1], Vn16[l + 1],
                bve[l + 1][None], bun[l + 1][None], bvn[l + 1][None])
            vxe_cur = vxe.reshape(B, V, H)
            e_state, agg4, den4, s1, s2 = edge_mid(
                e_state, s1, s2, gamma_e[l][None], beta_e[l][None],
                Ue16[l], be[l][None], vxe_prev,
                Ue16[l + 1], be[l + 1][None], vxe_cur,
                vx.reshape(B, V, H), s_mat)
            vxe_prev = vxe_cur
        else:
            x = node_update_last(
                x, ux, agg, den, gamma_n[l][None], beta_n[l][None])
            e_out = edge_final(
                e_state, s1, s2, gamma_e[l][None], beta_e[l][None],
                Ue16[l], be[l][None], vxe_prev)

    return x.reshape(B, V, H), e_out


# speedup vs baseline: 1.5894x; 1.5894x over previous
import functools

import jax
import jax.numpy as jnp
from jax.experimental import pallas as pl
from jax.experimental.pallas import tpu as pltpu

_EPS_BN = 1e-5
_EPS_DEN = 1e-20


def _node_init_body(coord_ref, wn_ref, ve_ref, un_ref, vn_ref,
                    bve_ref, bun_ref, bvn_ref,
                    x_ref, vxe_ref, ux_ref, vx_ref):
    f32 = jnp.float32
    coord = coord_ref[...].astype(jnp.bfloat16).astype(f32)
    wn = wn_ref[...].astype(jnp.bfloat16).astype(f32)
    nd = coord.shape[1]
    x = coord[:, 0:1] * wn[0][None, :]
    for d in range(1, nd):
        x = x + coord[:, d:d + 1] * wn[d][None, :]
    x_ref[...] = x
    x16 = x.astype(jnp.bfloat16)
    vxe_ref[...] = jnp.dot(x16, ve_ref[...], preferred_element_type=f32) + bve_ref[...]
    ux_ref[...] = jnp.dot(x16, un_ref[...], preferred_element_type=f32) + bun_ref[...]
    vx_ref[...] = jnp.dot(x16, vn_ref[...], preferred_element_type=f32) + bvn_ref[...]


def _node_update_core(x_ref, ux_ref, agg_ref, den_ref, gn_ref, bn_ref):
    t = ux_ref[...] + agg_ref[...] / (den_ref[...] + _EPS_DEN)
    mu = jnp.mean(t, axis=0, keepdims=True)
    var = jnp.mean((t - mu) ** 2, axis=0, keepdims=True)
    h = gn_ref[...] * (t - mu) * jax.lax.rsqrt(var + _EPS_BN) + bn_ref[...]
    return x_ref[...] + jnp.maximum(h, 0.0)


def _node_update_proj_body(x_ref, ux_ref, agg_ref, den_ref, gn_ref, bn_ref,
                           ve_ref, un_ref, vn_ref, bve_ref, bun_ref, bvn_ref,
                           xo_ref, vxe_ref, uxo_ref, vxo_ref):
    x = _node_update_core(x_ref, ux_ref, agg_ref, den_ref, gn_ref, bn_ref)
    xo_ref[...] = x
    f32 = jnp.float32
    x16 = x.astype(jnp.bfloat16)
    vxe_ref[...] = jnp.dot(x16, ve_ref[...], preferred_element_type=f32) + bve_ref[...]
    uxo_ref[...] = jnp.dot(x16, un_ref[...], preferred_element_type=f32) + bun_ref[...]
    vxo_ref[...] = jnp.dot(x16, vn_ref[...], preferred_element_type=f32) + bvn_ref[...]


def _node_update_last_body(x_ref, ux_ref, agg_ref, den_ref, gn_ref, bn_ref,
                           xo_ref):
    xo_ref[...] = _node_update_core(x_ref, ux_ref, agg_ref, den_ref,
                                    gn_ref, bn_ref)


def _edge_epilogue(etmp, vx_ref, agg_ref, den_ref, s1_ref, s2_ref):
    gate = jax.nn.sigmoid(etmp)
    vx_j = vx_ref[0]
    agg_ref[0, 0] = jnp.sum(gate * vx_j[None, :, :], axis=1)
    den_ref[0, 0] = jnp.sum(gate, axis=1)
    p1 = jnp.sum(etmp, axis=(0, 1))[None, :]
    p2 = jnp.sum(etmp * etmp, axis=(0, 1))[None, :]
    first = (pl.program_id(0) == 0) & (pl.program_id(1) == 0)

    @pl.when(first)
    def _():
        s1_ref[...] = p1
        s2_ref[...] = p2

    @pl.when(jnp.logical_not(first))
    def _():
        s1_ref[...] += p1
        s2_ref[...] += p2


def _edge_matmul(e_tile, u16_ref, bias_ref, vxe_ref, i0, ic_rows):
    icr, vv, hd = e_tile.shape
    m2 = jnp.dot(e_tile.reshape(icr * vv, hd).astype(jnp.bfloat16),
                 u16_ref[...],
                 preferred_element_type=jnp.float32).reshape(icr, vv, hd)
    vxe_i = vxe_ref[0, pl.ds(i0, ic_rows), :]
    vxe_j = vxe_ref[0] + bias_ref[...]
    return m2 + (vxe_i[:, None, :] + vxe_j[None, :, :])


def _bn_residual(e_tile, etmp, s1_ref, s2_ref, ge_ref, bte_ref, m_total):
    mu = s1_ref[0] / m_total
    var = s2_ref[0] / m_total - mu * mu
    inv = ge_ref[0] * jax.lax.rsqrt(var + _EPS_BN)
    shift = bte_ref[0] - mu * inv
    return e_tile + jnp.maximum(
        etmp * inv[None, None, :] + shift[None, None, :], 0.0)


def _edge_first_body(vals_ref, tags_ref, u16_ref, bias_ref, wvp_ref, ptab_ref,
                     vxe_ref, vx_ref,
                     e_ref, agg_ref, den_ref, s1_ref, s2_ref,
                     *, ic_rows, voc):
    i0 = pl.program_id(1) * ic_rows
    vals = vals_ref[0, pl.ds(i0, ic_rows), :]
    ncat = tags_ref.shape[3]
    e0 = vals[:, :, None] * wvp_ref[0][None, None, :]
    for c in range(ncat):
        t = tags_ref[0, pl.ds(i0, ic_rows), :, c]
        for v in range(voc):
            m = (t == v)[:, :, None]
            e0 = e0 + jnp.where(m, ptab_ref[c * voc + v][None, None, :], 0.0)
    e_ref[0] = e0
    etmp = _edge_matmul(e0, u16_ref, bias_ref, vxe_ref, i0, ic_rows)
    _edge_epilogue(etmp, vx_ref, agg_ref, den_ref, s1_ref, s2_ref)


def _edge_mid_body(e_ref, s1_ref, s2_ref, ge_ref, bte_ref,
                   up_ref, biasp_ref, vxep_ref,
                   uc_ref, biasc_ref, vxec_ref, vx_ref,
                   eo_ref, agg_ref, den_ref, s1o_ref, s2o_ref,
                   *, ic_rows, m_total):
    i0 = pl.program_id(1) * ic_rows
    e_prev = e_ref[0]
    etmp_prev = _edge_matmul(e_prev, up_ref, biasp_ref, vxep_ref, i0, ic_rows)
    e_new = _bn_residual(e_prev, etmp_prev, s1_ref, s2_ref, ge_ref, bte_ref,
                         m_total)
    eo_ref[0] = e_new
    etmp = _edge_matmul(e_new, uc_ref, biasc_ref, vxec_ref, i0, ic_rows)
    _edge_epilogue(etmp, vx_ref, agg_ref, den_ref, s1o_ref, s2o_ref)


def _edge_final_body(e_ref, s1_ref, s2_ref, ge_ref, bte_ref,
                     up_ref, biasp_ref, vxep_ref,
                     eo_ref, *, ic_rows, m_total):
    i0 = pl.program_id(1) * ic_rows
    e_prev = e_ref[0]
    etmp_prev = _edge_matmul(e_prev, up_ref, biasp_ref, vxep_ref, i0, ic_rows)
    eo_ref[0] = _bn_residual(e_prev, etmp_prev, s1_ref, s2_ref, ge_ref,
                             bte_ref, m_total)


def kernel(x_edges, x_edges_values, x_nodes_coord, W_nodes, W_vals, E_cat,
           Ue, Ve, Un, Vn, be, bve, bun, bvn,
           gamma_n, beta_n, gamma_e, beta_e):
    B, V, _, ncat = x_edges.shape
    H = W_nodes.shape[1]
    L = Ue.shape[0]
    voc = E_cat.shape[1]
    Hh = W_vals.shape[1]
    N = B * V
    f32 = jnp.float32

    IC = next(c for c in (32, 25, 20, 16, 10, 8, 5, 4, 2, 1) if V % c == 0)
    NIC = V // IC
    m_total = float(B * V * V)

    vals = x_edges_values.astype(f32)
    tags = x_edges.astype(jnp.int32)

    wvp = jnp.pad(W_vals[0], (0, H - Hh))[None, :]
    ptab = jnp.pad(E_cat, ((0, 0), (0, 0), (Hh, 0))).reshape(ncat * voc, H)
    Ue16 = Ue.astype(jnp.bfloat16)
    Ve16 = Ve.astype(jnp.bfloat16)
    Un16 = Un.astype(jnp.bfloat16)
    Vn16 = Vn.astype(jnp.bfloat16)

    cparams = pltpu.CompilerParams(
        dimension_semantics=("arbitrary", "arbitrary"))

    node_init = pl.pallas_call(
        _node_init_body,
        out_shape=[jax.ShapeDtypeStruct((N, H), f32)] * 4,
    )
    node_update_proj = pl.pallas_call(
        _node_update_proj_body,
        out_shape=[jax.ShapeDtypeStruct((N, H), f32)] * 4,
    )
    node_update_last = pl.pallas_call(
        _node_update_last_body,
        out_shape=jax.ShapeDtypeStruct((N, H), f32),
    )

    _rep = lambda b, i: (0, 0)
    _bvh = pl.BlockSpec((1, V, H), lambda b, i: (b, 0, 0))
    _tile = pl.BlockSpec((1, IC, V, H), lambda b, i: (b, i, 0, 0))
    _aggspec = pl.BlockSpec((1, 1, IC, H), lambda b, i: (b, i, 0, 0))
    _row = pl.BlockSpec((1, H), _rep)
    _wspec = pl.BlockSpec((H, H), _rep)
    edge_out_shapes = [
        jax.ShapeDtypeStruct((B, V, V, H), f32),
        jax.ShapeDtypeStruct((B, NIC, IC, H), f32),
        jax.ShapeDtypeStruct((B, NIC, IC, H), f32),
        jax.ShapeDtypeStruct((1, H), f32),
        jax.ShapeDtypeStruct((1, H), f32),
    ]
    edge_out_specs = [_tile, _aggspec, _aggspec, _row, _row]

    edge_first = pl.pallas_call(
        functools.partial(_edge_first_body, ic_rows=IC, voc=voc),
        grid=(B, NIC),
        in_specs=[
            pl.BlockSpec((1, V, V), lambda b, i: (b, 0, 0)),
            pl.BlockSpec((1, V, V, ncat), lambda b, i: (b, 0, 0, 0)),
            _wspec,
            _row,
            _row,
            pl.BlockSpec((ncat * voc, H), _rep),
            _bvh, _bvh,
        ],
        out_specs=edge_out_specs,
        out_shape=edge_out_shapes,
        compiler_params=cparams,
    )
    edge_mid = pl.pallas_call(
        functools.partial(_edge_mid_body, ic_rows=IC, m_total=m_total),
        grid=(B, NIC),
        in_specs=[_tile, _row, _row, _row, _row,
                  _wspec, _row, _bvh,
                  _wspec, _row, _bvh, _bvh],
        out_specs=edge_out_specs,
        out_shape=edge_out_shapes,
        compiler_params=cparams,
    )
    edge_final = pl.pallas_call(
        functools.partial(_edge_final_body, ic_rows=IC, m_total=m_total),
        grid=(B, NIC),
        in_specs=[_tile, _row, _row, _row, _row,
                  _wspec, _row, _bvh],
        out_specs=_tile,
        out_shape=jax.ShapeDtypeStruct((B, V, V, H), f32),
        compiler_params=cparams,
    )

    coord = x_nodes_coord.reshape(N, -1).astype(f32)
    x, vxe, ux, vx = node_init(
        coord, W_nodes, Ve16[0], Un16[0], Vn16[0],
        bve[0][None], bun[0][None], bvn[0][None])

    vxe_prev = vxe.reshape(B, V, H)
    e_state, agg4, den4, s1, s2 = edge_first(
        vals, tags, Ue16[0], be[0][None], wvp, ptab,
        vxe_prev, vx.reshape(B, V, H))

    for l in range(L):
        agg = agg4.reshape(N, H)
        den = den4.reshape(N, H)
        if l < L - 1:
            x, vxe, ux, vx = node_update_proj(
                x, ux, agg, den, gamma_n[l][None], beta_n[l][None],
                Ve16[l + 1], Un16[l + 1], Vn16[l + 1],
                bve[l + 1][None], bun[l + 1][None], bvn[l + 1][None])
            vxe_cur = vxe.reshape(B, V, H)
            e_state, agg4, den4, s1, s2 = edge_mid(
                e_state, s1, s2, gamma_e[l][None], beta_e[l][None],
                Ue16[l], be[l][None], vxe_prev,
                Ue16[l + 1], be[l + 1][None], vxe_cur,
                vx.reshape(B, V, H))
            vxe_prev = vxe_cur
        else:
            x = node_update_last(
                x, ux, agg, den, gamma_n[l][None], beta_n[l][None])
            e_out = edge_final(
                e_state, s1, s2, gamma_e[l][None], beta_e[l][None],
                Ue16[l], be[l][None], vxe_prev)

    return x.reshape(B, V, H), e_out
